# baseline (speedup 1.0000x reference)
import functools
import math

import jax
import jax.numpy as jnp
from jax import lax
from jax.experimental import pallas as pl
from jax.experimental.pallas import tpu as pltpu
from jax.experimental.pallas import tpu_sc as plsc

_UNITS = 1000000
_NEG = 8192
_BATCH = 4096
_DIM = 64
_BB = 512

_LOG_U1 = math.log(float(_UNITS) + 1.0)
_LOG_CONST = math.log(float(_NEG)) - math.log(_LOG_U1)

_IDXW = 128


def _sc_gather_body(table_hbm, bias_hbm, sampled2_hbm, targets2_hbm,
                    sw_out, tw_out, sb_out, tb_out,
                    sidx, tidx, srows, trows, sbias, tbias, sem):
    nc = 2
    wid = lax.axis_index("s") * nc + lax.axis_index("c")
    s_chunks = 2
    t_chunks = 1
    s_per_w = s_chunks * _IDXW
    t_per_w = t_chunks * _IDXW

    pltpu.sync_copy(sampled2_hbm.at[pl.ds(wid * s_chunks, s_chunks)], sidx)
    pltpu.sync_copy(targets2_hbm.at[pl.ds(wid * t_chunks, t_chunks)], tidx)

    cps = []
    for c in range(s_chunks):
        cps.append(pltpu.async_copy(
            table_hbm.at[sidx.at[c]], srows.at[pl.ds(c * _IDXW, _IDXW)], sem))
        cps.append(pltpu.async_copy(
            bias_hbm.at[sidx.at[c]], sbias.at[pl.ds(c * _IDXW, _IDXW)], sem))
    for c in range(t_chunks):
        cps.append(pltpu.async_copy(
            table_hbm.at[tidx.at[c]], trows.at[pl.ds(c * _IDXW, _IDXW)], sem))
        cps.append(pltpu.async_copy(
            bias_hbm.at[tidx.at[c]], tbias.at[pl.ds(c * _IDXW, _IDXW)], sem))
    for cp in cps:
        cp.wait()

    pltpu.sync_copy(srows, sw_out.at[pl.ds(wid * s_per_w, s_per_w)])
    pltpu.sync_copy(trows, tw_out.at[pl.ds(wid * t_per_w, t_per_w)])
    pltpu.sync_copy(sbias, sb_out.at[pl.ds(wid * s_per_w, s_per_w)])
    pltpu.sync_copy(tbias, tb_out.at[pl.ds(wid * t_per_w, t_per_w)])


def _sc_gather(table, bias, sampled2, targets2):
    mesh = plsc.VectorSubcoreMesh(core_axis_name="c", subcore_axis_name="s")
    fn = functools.partial(
        pl.kernel, mesh=mesh,
        compiler_params=pltpu.CompilerParams(use_tc_tiling_on_sc=False),
        out_type=[
            jax.ShapeDtypeStruct((_NEG, _DIM), jnp.float32),
            jax.ShapeDtypeStruct((_BATCH, _DIM), jnp.float32),
            jax.ShapeDtypeStruct((_NEG,), jnp.float32),
            jax.ShapeDtypeStruct((_BATCH,), jnp.float32),
        ],
        scratch_types=[
            pltpu.VMEM((2, _IDXW), jnp.int32),
            pltpu.VMEM((1, _IDXW), jnp.int32),
            pltpu.VMEM((256, _DIM), jnp.float32),
            pltpu.VMEM((128, _DIM), jnp.float32),
            pltpu.VMEM((256,), jnp.float32),
            pltpu.VMEM((128,), jnp.float32),
            pltpu.SemaphoreType.DMA,
        ],
    )(_sc_gather_body)
    return fn(table, bias, sampled2, targets2)


def _loss_body(x_ref, tw_ref, tb_ref, tid_ref, sw_ref, sb_ref, sid_ref, out_ref):
    i = pl.program_id(0)
    x = x_ref[...]
    tw = tw_ref[...]
    tb = tb_ref[...]
    tid = tid_ref[...]
    sw = sw_ref[...]
    sb = sb_ref[...]
    sid = sid_ref[...]

    tidf = tid.astype(jnp.float32)
    log_np_t = jnp.log(jnp.log(tidf + 2.0) - jnp.log(tidf + 1.0)) + _LOG_CONST
    true_logits = (jnp.sum(x * tw, axis=1, keepdims=True) + tb - log_np_t)

    sidf = sid.astype(jnp.float32)
    log_np_s = jnp.log(jnp.log(sidf + 2.0) - jnp.log(sidf + 1.0)) + _LOG_CONST
    sl = lax.dot_general(x, sw, (((1,), (1,)), ((), ())),
                         preferred_element_type=jnp.float32)
    sl = sl + (sb - log_np_s)
    sl = jnp.where(tid == sid, sl - 1e9, sl)

    m = jnp.maximum(jnp.max(sl, axis=1, keepdims=True), true_logits)
    se = jnp.sum(jnp.exp(sl - m), axis=1, keepdims=True) + jnp.exp(true_logits - m)
    per_ex = jnp.log(se) + m - true_logits
    part = jnp.sum(per_ex) * (1.0 / _BATCH)

    @pl.when(i == 0)
    def _():
        out_ref[0, 0] = 0.0

    out_ref[0, 0] += part


def _loss(logits, tw, tb2, tid2, sw, sb2, sid2):
    return pl.pallas_call(
        _loss_body,
        grid=(_BATCH // _BB,),
        in_specs=[
            pl.BlockSpec((_BB, _DIM), lambda i: (i, 0)),
            pl.BlockSpec((_BB, _DIM), lambda i: (i, 0)),
            pl.BlockSpec((_BB, 1), lambda i: (i, 0)),
            pl.BlockSpec((_BB, 1), lambda i: (i, 0)),
            pl.BlockSpec((_NEG, _DIM), lambda i: (0, 0)),
            pl.BlockSpec((1, _NEG), lambda i: (0, 0)),
            pl.BlockSpec((1, _NEG), lambda i: (0, 0)),
        ],
        out_specs=pl.BlockSpec(memory_space=pltpu.SMEM),
        out_shape=jax.ShapeDtypeStruct((1, 1), jnp.float32),
    )(logits, tw, tb2, tid2, sw, sb2, sid2)


def kernel(logits, targets, kernel, bias, sampled):
    sampled2 = sampled.reshape(_NEG // _IDXW, _IDXW)
    targets2 = targets.reshape(_BATCH // _IDXW, _IDXW)
    sw, tw, sb, tb = _sc_gather(kernel, bias, sampled2, targets2)
    out = _loss(logits, tw, tb.reshape(_BATCH, 1), targets.reshape(_BATCH, 1),
                sw, sb.reshape(1, _NEG), sampled.reshape(1, _NEG))
    return out[0, 0]
